# Initial kernel scaffold; baseline (speedup 1.0000x reference)
import jax
import jax.numpy as jnp
from jax.experimental import pallas as pl


def kernel(x, w1, b1, w2, b2, wf1, bf1, wf2, bf2):
    raise NotImplementedError("write your pallas kernel here")


# speedup vs baseline: 1.8608x; 1.8608x over previous
import jax
import jax.numpy as jnp
from jax.experimental import pallas as pl
from jax.experimental.pallas import tpu as pltpu

TB = 16


def _conv_stack_kernel(x_ref, w1_ref, b1_ref, w2_ref, b2_ref, o_ref,
                       a1_scr, y_scr):
    x = x_ref[...]

    acc = jnp.zeros((TB, 26, 32, 32), jnp.float32)
    for di in range(3):
        for dj in range(3):
            acc = acc + (x[:, di:di + 26, dj:dj + 32, None]
                         * w1_ref[di * 3 + dj][None, None, None, :])
    a1_scr[:, :, 0:32, :] = jnp.maximum(acc + b1_ref[0], 0.0)
    a1_scr[:, :, 32:40, :] = jnp.zeros((TB, 26, 8, 32), jnp.float32)

    accy = jnp.zeros((TB * 192, 256), jnp.float32)
    for di in range(3):
        patch = jnp.concatenate(
            [a1_scr[:, pl.ds(di, 24), pl.ds(pos, 8, 4), :] for pos in range(6)],
            axis=-1)
        accy = accy + jnp.dot(patch.reshape(TB * 192, 192), w2_ref[di],
                              preferred_element_type=jnp.float32)
    y = jnp.maximum(accy + b2_ref[0], 0.0)
    y_scr[...] = y.reshape(TB, 24, 8, 256)

    p0 = jnp.maximum(y_scr[..., 0:64], y_scr[..., 64:128])
    p1 = jnp.maximum(y_scr[..., 128:192], y_scr[..., 192:256])
    p0 = p0.reshape(TB, 12, 2, 8, 64)
    p1 = p1.reshape(TB, 12, 2, 8, 64)
    v0 = jnp.maximum(p0[:, :, 0], p0[:, :, 1])
    v1 = jnp.maximum(p1[:, :, 0], p1[:, :, 1])
    o_ref[:, :, :, 0, :] = v0[:, :, 0:6, :]
    o_ref[:, :, :, 1, :] = v1[:, :, 0:6, :]


def _fc_head_kernel(f_ref, wa_ref, ba_ref, wb_ref, bb_ref, o_ref):
    h = jnp.dot(f_ref[...], wa_ref[...], preferred_element_type=jnp.float32)
    h = jnp.maximum(h + ba_ref[...], 0.0)
    z = jnp.dot(h, wb_ref[...], preferred_element_type=jnp.float32) + bb_ref[...]
    m = jnp.max(z, axis=1, keepdims=True)
    s = z - m
    o_ref[...] = s - jnp.log(jnp.sum(jnp.exp(s), axis=1, keepdims=True))


def kernel(x, w1, b1, w2, b2, wf1, bf1, wf2, bf2):
    B = x.shape[0]
    f32 = jnp.float32

    xp = jnp.pad(x.reshape(B, 28, 28).astype(f32), ((0, 0), (0, 0), (0, 8)))
    w1k = jnp.transpose(w1.astype(f32), (2, 3, 1, 0)).reshape(9, 32)
    b1r = b1.reshape(1, 32).astype(f32)
    w2k = jnp.transpose(w2.astype(f32), (2, 3, 1, 0))
    wg = jnp.zeros((3, 6, 32, 4, 64), f32)
    for wo in range(4):
        for dj in range(3):
            wg = wg.at[:, wo + dj, :, wo, :].set(w2k[:, dj])
    wg = wg.reshape(3, 192, 256)
    b2t = jnp.tile(b2.reshape(1, 64).astype(f32), (1, 4))

    wa = wf1.astype(f32).reshape(128, 64, 12, 12).transpose(2, 3, 1, 0).reshape(9216, 128)
    ba = bf1.reshape(1, 128).astype(f32)
    wb = jnp.zeros((128, 128), f32).at[:, :10].set(wf2.astype(f32).T)
    bb = jnp.full((1, 128), -1e30, f32).at[0, :10].set(bf2.astype(f32))

    pooled = pl.pallas_call(
        _conv_stack_kernel,
        out_shape=jax.ShapeDtypeStruct((B, 12, 6, 2, 64), f32),
        grid=(B // TB,),
        in_specs=[
            pl.BlockSpec((TB, 28, 36), lambda i: (i, 0, 0)),
            pl.BlockSpec((9, 32), lambda i: (0, 0)),
            pl.BlockSpec((1, 32), lambda i: (0, 0)),
            pl.BlockSpec((3, 192, 256), lambda i: (0, 0, 0)),
            pl.BlockSpec((1, 256), lambda i: (0, 0)),
        ],
        out_specs=pl.BlockSpec((TB, 12, 6, 2, 64), lambda i: (i, 0, 0, 0, 0)),
        scratch_shapes=[pltpu.VMEM((TB, 26, 40, 32), f32),
                        pltpu.VMEM((TB, 24, 8, 256), f32)],
        compiler_params=pltpu.CompilerParams(dimension_semantics=("parallel",)),
    )(xp, w1k, b1r, wg, b2t)

    feats = pooled.reshape(B, 9216)

    BM = min(256, B)
    out = pl.pallas_call(
        _fc_head_kernel,
        out_shape=jax.ShapeDtypeStruct((B, 128), f32),
        grid=(B // BM,),
        in_specs=[
            pl.BlockSpec((BM, 9216), lambda i: (i, 0)),
            pl.BlockSpec((9216, 128), lambda i: (0, 0)),
            pl.BlockSpec((1, 128), lambda i: (0, 0)),
            pl.BlockSpec((128, 128), lambda i: (0, 0)),
            pl.BlockSpec((1, 128), lambda i: (0, 0)),
        ],
        out_specs=pl.BlockSpec((BM, 128), lambda i: (i, 0)),
        compiler_params=pltpu.CompilerParams(dimension_semantics=("parallel",)),
    )(feats, wa, ba, wb, bb)

    return out[:, :10]


# speedup vs baseline: 2.1662x; 1.1641x over previous
import jax
import jax.numpy as jnp
from jax.experimental import pallas as pl
from jax.experimental.pallas import tpu as pltpu

TB = 16


def _conv_stack_kernel(x_ref, w1_ref, b1_ref, w2_ref, b2_ref, o_ref,
                       a1_scr, y_scr):
    x = x_ref[...]

    acc = jnp.zeros((TB, 26, 32, 32), jnp.float32)
    for di in range(3):
        for dj in range(3):
            acc = acc + (x[:, di:di + 26, dj:dj + 32, None]
                         * w1_ref[di * 3 + dj][None, None, None, :])
    a1_scr[:, :, 0:32, :] = jnp.maximum(acc + b1_ref[0], 0.0)
    a1_scr[:, :, 32:40, :] = jnp.zeros((TB, 26, 8, 32), jnp.float32)

    pfull = jnp.concatenate(
        [a1_scr[:, :, pl.ds(pos, 8, 4), :] for pos in range(6)],
        axis=-1)
    accy = jnp.zeros((TB * 192, 256), jnp.float32)
    for di in range(3):
        patch = pfull[:, di:di + 24]
        accy = accy + jnp.dot(patch.reshape(TB * 192, 192), w2_ref[di],
                              preferred_element_type=jnp.float32)
    y = jnp.maximum(accy + b2_ref[0], 0.0)
    y_scr[...] = y.reshape(TB, 24, 8, 256)

    p0 = jnp.maximum(y_scr[..., 0:64], y_scr[..., 64:128])
    p1 = jnp.maximum(y_scr[..., 128:192], y_scr[..., 192:256])
    p0 = p0.reshape(TB, 12, 2, 8, 64)
    p1 = p1.reshape(TB, 12, 2, 8, 64)
    v0 = jnp.maximum(p0[:, :, 0], p0[:, :, 1])
    v1 = jnp.maximum(p1[:, :, 0], p1[:, :, 1])
    o_ref[:, :, :, 0, :] = v0[:, :, 0:6, :]
    o_ref[:, :, :, 1, :] = v1[:, :, 0:6, :]


def _fc_head_kernel(f_ref, wa_ref, ba_ref, wb_ref, bb_ref, o_ref):
    h = jnp.dot(f_ref[...], wa_ref[...], preferred_element_type=jnp.float32)
    h = jnp.maximum(h + ba_ref[...], 0.0)
    z = jnp.dot(h, wb_ref[...], preferred_element_type=jnp.float32) + bb_ref[...]
    m = jnp.max(z, axis=1, keepdims=True)
    s = z - m
    o_ref[...] = s - jnp.log(jnp.sum(jnp.exp(s), axis=1, keepdims=True))


def kernel(x, w1, b1, w2, b2, wf1, bf1, wf2, bf2):
    B = x.shape[0]
    f32 = jnp.float32

    xp = jnp.pad(x.reshape(B, 28, 28).astype(f32), ((0, 0), (0, 0), (0, 8)))
    w1k = jnp.transpose(w1.astype(f32), (2, 3, 1, 0)).reshape(9, 32)
    b1r = b1.reshape(1, 32).astype(f32)
    w2k = jnp.transpose(w2.astype(f32), (2, 3, 1, 0))
    wg = jnp.zeros((3, 6, 32, 4, 64), f32)
    for wo in range(4):
        for dj in range(3):
            wg = wg.at[:, wo + dj, :, wo, :].set(w2k[:, dj])
    wg = wg.reshape(3, 192, 256)
    b2t = jnp.tile(b2.reshape(1, 64).astype(f32), (1, 4))

    wa = wf1.astype(f32).reshape(128, 64, 12, 12).transpose(2, 3, 1, 0).reshape(9216, 128)
    ba = bf1.reshape(1, 128).astype(f32)
    wb = jnp.zeros((128, 128), f32).at[:, :10].set(wf2.astype(f32).T)
    bb = jnp.full((1, 128), -1e30, f32).at[0, :10].set(bf2.astype(f32))

    pooled = pl.pallas_call(
        _conv_stack_kernel,
        out_shape=jax.ShapeDtypeStruct((B, 12, 6, 2, 64), f32),
        grid=(B // TB,),
        in_specs=[
            pl.BlockSpec((TB, 28, 36), lambda i: (i, 0, 0)),
            pl.BlockSpec((9, 32), lambda i: (0, 0)),
            pl.BlockSpec((1, 32), lambda i: (0, 0)),
            pl.BlockSpec((3, 192, 256), lambda i: (0, 0, 0)),
            pl.BlockSpec((1, 256), lambda i: (0, 0)),
        ],
        out_specs=pl.BlockSpec((TB, 12, 6, 2, 64), lambda i: (i, 0, 0, 0, 0)),
        scratch_shapes=[pltpu.VMEM((TB, 26, 40, 32), f32),
                        pltpu.VMEM((TB, 24, 8, 256), f32)],
        compiler_params=pltpu.CompilerParams(dimension_semantics=("parallel",)),
    )(xp, w1k, b1r, wg, b2t)

    feats = pooled.reshape(B, 9216)

    BM = min(256, B)
    out = pl.pallas_call(
        _fc_head_kernel,
        out_shape=jax.ShapeDtypeStruct((B, 128), f32),
        grid=(B // BM,),
        in_specs=[
            pl.BlockSpec((BM, 9216), lambda i: (i, 0)),
            pl.BlockSpec((9216, 128), lambda i: (0, 0)),
            pl.BlockSpec((1, 128), lambda i: (0, 0)),
            pl.BlockSpec((128, 128), lambda i: (0, 0)),
            pl.BlockSpec((1, 128), lambda i: (0, 0)),
        ],
        out_specs=pl.BlockSpec((BM, 128), lambda i: (i, 0)),
        compiler_params=pltpu.CompilerParams(dimension_semantics=("parallel",)),
    )(feats, wa, ba, wb, bb)

    return out[:, :10]


# speedup vs baseline: 2.9501x; 1.3619x over previous
import jax
import jax.numpy as jnp
from jax.experimental import pallas as pl
from jax.experimental.pallas import tpu as pltpu

TB = 16


def _conv_stack_kernel(x_ref, w1_ref, b1_ref, w2_ref, b2_ref, o_ref,
                       xrep_scr, a1_scr, y_scr):
    x = x_ref[...]

    xrep_scr[:, :, 0:36, :] = jnp.broadcast_to(x[..., None], (TB, 28, 36, 32))
    xrep_scr[:, :, 36:40, :] = jnp.zeros((TB, 28, 4, 32), jnp.float32)
    prods = [xrep_scr[:, di:di + 26, dj:dj + 32, :]
             * w1_ref[di * 3 + dj][None, None, None, :]
             for di in range(3) for dj in range(3)]
    while len(prods) > 1:
        prods = [a + b for a, b in zip(prods[::2], prods[1::2])] + \
                ([prods[-1]] if len(prods) % 2 else [])
    a1_scr[:, :, 0:32, :] = jnp.maximum(prods[0] + b1_ref[0], 0.0)
    a1_scr[:, :, 32:40, :] = jnp.zeros((TB, 26, 8, 32), jnp.float32)

    pfull = jnp.concatenate(
        [a1_scr[:, :, pl.ds(pos, 8, 4), :] for pos in range(6)],
        axis=-1)
    accy = jnp.zeros((TB * 192, 256), jnp.float32)
    for di in range(3):
        patch = pfull[:, di:di + 24]
        accy = accy + jnp.dot(patch.reshape(TB * 192, 192), w2_ref[di],
                              preferred_element_type=jnp.float32)
    y = jnp.maximum(accy + b2_ref[0], 0.0)
    y_scr[...] = y.reshape(TB, 24, 8, 256)

    p0 = jnp.maximum(y_scr[..., 0:64], y_scr[..., 64:128])
    p1 = jnp.maximum(y_scr[..., 128:192], y_scr[..., 192:256])
    p0 = p0.reshape(TB, 12, 2, 8, 64)
    p1 = p1.reshape(TB, 12, 2, 8, 64)
    v0 = jnp.maximum(p0[:, :, 0], p0[:, :, 1])
    v1 = jnp.maximum(p1[:, :, 0], p1[:, :, 1])
    o_ref[:, :, :, 0, :] = v0[:, :, 0:6, :]
    o_ref[:, :, :, 1, :] = v1[:, :, 0:6, :]


def _fc_head_kernel(f_ref, wa_ref, ba_ref, wb_ref, bb_ref, o_ref):
    h = jnp.dot(f_ref[...], wa_ref[...], preferred_element_type=jnp.float32)
    h = jnp.maximum(h + ba_ref[...], 0.0)
    z = jnp.dot(h, wb_ref[...], preferred_element_type=jnp.float32) + bb_ref[...]
    m = jnp.max(z, axis=1, keepdims=True)
    s = z - m
    o_ref[...] = s - jnp.log(jnp.sum(jnp.exp(s), axis=1, keepdims=True))


def kernel(x, w1, b1, w2, b2, wf1, bf1, wf2, bf2):
    B = x.shape[0]
    f32 = jnp.float32

    xp = jnp.pad(x.reshape(B, 28, 28).astype(f32), ((0, 0), (0, 0), (0, 8)))
    w1k = jnp.transpose(w1.astype(f32), (2, 3, 1, 0)).reshape(9, 32)
    b1r = b1.reshape(1, 32).astype(f32)
    w2k = jnp.transpose(w2.astype(f32), (2, 3, 1, 0))
    wg = jnp.zeros((3, 6, 32, 4, 64), f32)
    for wo in range(4):
        for dj in range(3):
            wg = wg.at[:, wo + dj, :, wo, :].set(w2k[:, dj])
    wg = wg.reshape(3, 192, 256)
    b2t = jnp.tile(b2.reshape(1, 64).astype(f32), (1, 4))

    wa = wf1.astype(f32).reshape(128, 64, 12, 12).transpose(2, 3, 1, 0).reshape(9216, 128)
    ba = bf1.reshape(1, 128).astype(f32)
    wb = jnp.zeros((128, 128), f32).at[:, :10].set(wf2.astype(f32).T)
    bb = jnp.full((1, 128), -1e30, f32).at[0, :10].set(bf2.astype(f32))

    pooled = pl.pallas_call(
        _conv_stack_kernel,
        out_shape=jax.ShapeDtypeStruct((B, 12, 6, 2, 64), f32),
        grid=(B // TB,),
        in_specs=[
            pl.BlockSpec((TB, 28, 36), lambda i: (i, 0, 0)),
            pl.BlockSpec((9, 32), lambda i: (0, 0)),
            pl.BlockSpec((1, 32), lambda i: (0, 0)),
            pl.BlockSpec((3, 192, 256), lambda i: (0, 0, 0)),
            pl.BlockSpec((1, 256), lambda i: (0, 0)),
        ],
        out_specs=pl.BlockSpec((TB, 12, 6, 2, 64), lambda i: (i, 0, 0, 0, 0)),
        scratch_shapes=[pltpu.VMEM((TB, 28, 40, 32), f32),
                        pltpu.VMEM((TB, 26, 40, 32), f32),
                        pltpu.VMEM((TB, 24, 8, 256), f32)],
        compiler_params=pltpu.CompilerParams(dimension_semantics=("parallel",)),
    )(xp, w1k, b1r, wg, b2t)

    feats = pooled.reshape(B, 9216)

    BM = min(256, B)
    out = pl.pallas_call(
        _fc_head_kernel,
        out_shape=jax.ShapeDtypeStruct((B, 128), f32),
        grid=(B // BM,),
        in_specs=[
            pl.BlockSpec((BM, 9216), lambda i: (i, 0)),
            pl.BlockSpec((9216, 128), lambda i: (0, 0)),
            pl.BlockSpec((1, 128), lambda i: (0, 0)),
            pl.BlockSpec((128, 128), lambda i: (0, 0)),
            pl.BlockSpec((1, 128), lambda i: (0, 0)),
        ],
        out_specs=pl.BlockSpec((BM, 128), lambda i: (i, 0)),
        compiler_params=pltpu.CompilerParams(dimension_semantics=("parallel",)),
    )(feats, wa, ba, wb, bb)

    return out[:, :10]


# speedup vs baseline: 4.8406x; 1.6408x over previous
import jax
import jax.numpy as jnp
from jax.experimental import pallas as pl
from jax.experimental.pallas import tpu as pltpu

TB = 16


def _conv_stack_kernel(x_ref, w1_ref, b1_ref, w2_ref, b2_ref, o_ref,
                       pf_scr, y_scr):
    xv = x_ref[...]

    xcat = jnp.concatenate([xv[:, di:di + 32, :] for di in range(3)],
                           axis=-1)
    a1 = jnp.dot(xcat.reshape(TB * 32, 384), w1_ref[...],
                 preferred_element_type=jnp.float32)
    a1 = jnp.maximum(a1 + b1_ref[...], 0.0).reshape(TB, 32, 1024)

    for wg in range(8):
        off = 128 * wg if wg < 7 else 768
        pf_scr[:, :, wg, :] = a1[:, 0:26, off:off + 192]

    accy = jnp.zeros((TB * 192, 256), jnp.float32)
    for di in range(3):
        patch = pf_scr[:, pl.ds(di, 24), :, :]
        accy = accy + jnp.dot(patch.reshape(TB * 192, 192), w2_ref[di],
                              preferred_element_type=jnp.float32)
    y = jnp.maximum(accy + b2_ref[0], 0.0)
    y_scr[...] = y.reshape(TB, 24, 8, 256)

    p0 = jnp.maximum(y_scr[..., 0:64], y_scr[..., 64:128])
    p1 = jnp.maximum(y_scr[..., 128:192], y_scr[..., 192:256])
    p0 = p0.reshape(TB, 12, 2, 8, 64)
    p1 = p1.reshape(TB, 12, 2, 8, 64)
    v0 = jnp.maximum(p0[:, :, 0], p0[:, :, 1])
    v1 = jnp.maximum(p1[:, :, 0], p1[:, :, 1])
    o_ref[:, :, :, 0, :] = v0[:, :, 0:6, :]
    o_ref[:, :, :, 1, :] = v1[:, :, 0:6, :]


def _fc_head_kernel(f_ref, wa_ref, ba_ref, wb_ref, bb_ref, o_ref):
    h = jnp.dot(f_ref[...], wa_ref[...], preferred_element_type=jnp.float32)
    h = jnp.maximum(h + ba_ref[...], 0.0)
    z = jnp.dot(h, wb_ref[...], preferred_element_type=jnp.float32) + bb_ref[...]
    m = jnp.max(z, axis=1, keepdims=True)
    s = z - m
    o_ref[...] = s - jnp.log(jnp.sum(jnp.exp(s), axis=1, keepdims=True))


def kernel(x, w1, b1, w2, b2, wf1, bf1, wf2, bf2):
    B = x.shape[0]
    f32 = jnp.float32

    xp = jnp.pad(x.reshape(B, 28, 28).astype(f32),
                 ((0, 0), (0, 8), (0, 100)))

    w1k = jnp.transpose(w1.astype(f32), (2, 3, 1, 0)).reshape(3, 3, 32)
    eyes = jnp.stack([jnp.eye(128, 32, -dj, dtype=f32) for dj in range(3)])
    w1big = jnp.einsum('jab,ijc->iabc', eyes, w1k).reshape(384, 1024)
    b1big = jnp.tile(b1.reshape(1, 32).astype(f32), (1, 32))

    w2k = jnp.transpose(w2.astype(f32), (2, 3, 1, 0))
    wg = jnp.zeros((3, 6, 32, 4, 64), f32)
    for wo in range(4):
        for dj in range(3):
            wg = wg.at[:, wo + dj, :, wo, :].set(w2k[:, dj])
    wg = wg.reshape(3, 192, 256)
    b2t = jnp.tile(b2.reshape(1, 64).astype(f32), (1, 4))

    wa = wf1.astype(f32).reshape(128, 64, 12, 12).transpose(2, 3, 1, 0).reshape(9216, 128)
    ba = bf1.reshape(1, 128).astype(f32)
    wb = jnp.zeros((128, 128), f32).at[:, :10].set(wf2.astype(f32).T)
    bb = jnp.full((1, 128), -1e30, f32).at[0, :10].set(bf2.astype(f32))

    pooled = pl.pallas_call(
        _conv_stack_kernel,
        out_shape=jax.ShapeDtypeStruct((B, 12, 6, 2, 64), f32),
        grid=(B // TB,),
        in_specs=[
            pl.BlockSpec((TB, 36, 128), lambda i: (i, 0, 0)),
            pl.BlockSpec((384, 1024), lambda i: (0, 0)),
            pl.BlockSpec((1, 1024), lambda i: (0, 0)),
            pl.BlockSpec((3, 192, 256), lambda i: (0, 0, 0)),
            pl.BlockSpec((1, 256), lambda i: (0, 0)),
        ],
        out_specs=pl.BlockSpec((TB, 12, 6, 2, 64), lambda i: (i, 0, 0, 0, 0)),
        scratch_shapes=[pltpu.VMEM((TB, 26, 8, 192), f32),
                        pltpu.VMEM((TB, 24, 8, 256), f32)],
        compiler_params=pltpu.CompilerParams(dimension_semantics=("parallel",)),
    )(xp, w1big, b1big, wg, b2t)

    feats = pooled.reshape(B, 9216)

    BM = min(256, B)
    out = pl.pallas_call(
        _fc_head_kernel,
        out_shape=jax.ShapeDtypeStruct((B, 128), f32),
        grid=(B // BM,),
        in_specs=[
            pl.BlockSpec((BM, 9216), lambda i: (i, 0)),
            pl.BlockSpec((9216, 128), lambda i: (0, 0)),
            pl.BlockSpec((1, 128), lambda i: (0, 0)),
            pl.BlockSpec((128, 128), lambda i: (0, 0)),
            pl.BlockSpec((1, 128), lambda i: (0, 0)),
        ],
        out_specs=pl.BlockSpec((BM, 128), lambda i: (i, 0)),
        compiler_params=pltpu.CompilerParams(dimension_semantics=("parallel",)),
    )(feats, wa, ba, wb, bb)

    return out[:, :10]
